# baseline (speedup 1.0000x reference)
import functools

import jax
import jax.numpy as jnp
from jax import lax
from jax.experimental import pallas as pl
from jax.experimental.pallas import tpu as pltpu
from jax.experimental.pallas import tpu_sc as plsc

_BATCH = 16384
_HIDDEN = 128
_NUM_WORKERS = 32
_ROWS_PER_WORKER = _BATCH // _NUM_WORKERS
_CHUNK = 128
_NUM_CHUNKS = _ROWS_PER_WORKER // _CHUNK


def _make_embed(num_rows: int):
  mesh = plsc.VectorSubcoreMesh(core_axis_name="c", subcore_axis_name="s")

  @functools.partial(
      pl.kernel,
      mesh=mesh,
      out_type=jax.ShapeDtypeStruct((_BATCH, _HIDDEN), jnp.float32),
      scratch_types=[
          pltpu.VMEM((_NUM_CHUNKS, _CHUNK), jnp.int32),
          pltpu.VMEM((_ROWS_PER_WORKER, _HIDDEN), jnp.float32),
          pltpu.SemaphoreType.DMA((_NUM_CHUNKS,)),
          pltpu.SemaphoreType.DMA((_NUM_CHUNKS,)),
      ],
  )
  def embed(labels_hbm, table_hbm, out_hbm, idx_v, rows_v, gsem, ssem):
    wid = lax.axis_index("s") * 2 + lax.axis_index("c")
    base = wid * _ROWS_PER_WORKER
    pltpu.sync_copy(labels_hbm.at[wid], idx_v)
    gathers = []
    for j in range(_NUM_CHUNKS):
      gathers.append(
          pltpu.async_copy(
              table_hbm.at[idx_v.at[j]],
              rows_v.at[pl.ds(j * _CHUNK, _CHUNK)],
              gsem.at[j],
          ))
    writes = []
    for j in range(_NUM_CHUNKS):
      gathers[j].wait()
      writes.append(
          pltpu.async_copy(
              rows_v.at[pl.ds(j * _CHUNK, _CHUNK)],
              out_hbm.at[pl.ds(base + j * _CHUNK, _CHUNK)],
              ssem.at[j],
          ))
    for w in writes:
      w.wait()

  return embed


def kernel(labels, table):
  labels3 = labels.astype(jnp.int32).reshape(
      _NUM_WORKERS, _NUM_CHUNKS, _CHUNK)
  return _make_embed(table.shape[0])(labels3, table)


# speedup vs baseline: 1.0156x; 1.0156x over previous
import functools

import jax
import jax.numpy as jnp
from jax import lax
from jax.experimental import pallas as pl
from jax.experimental.pallas import tpu as pltpu
from jax.experimental.pallas import tpu_sc as plsc

_BATCH = 16384
_HIDDEN = 128
_NUM_WORKERS = 32
_ROWS_PER_WORKER = _BATCH // _NUM_WORKERS
_CHUNK = 128
_NUM_CHUNKS = _ROWS_PER_WORKER // _CHUNK


def _make_embed(num_rows: int):
  mesh = plsc.VectorSubcoreMesh(core_axis_name="c", subcore_axis_name="s")

  @functools.partial(
      pl.kernel,
      mesh=mesh,
      out_type=jax.ShapeDtypeStruct((_BATCH, _HIDDEN), jnp.float32),
      scratch_types=[
          pltpu.VMEM((_ROWS_PER_WORKER,), jnp.int32),
          pltpu.VMEM((_ROWS_PER_WORKER, _HIDDEN), jnp.float32),
          pltpu.SemaphoreType.DMA,
      ],
  )
  def embed(labels_hbm, table_hbm, out_hbm, idx_v, rows_v, sem):
    wid = lax.axis_index("s") * 2 + lax.axis_index("c")
    base = wid * _ROWS_PER_WORKER
    pltpu.sync_copy(labels_hbm.at[pl.ds(base, _ROWS_PER_WORKER)], idx_v)
    pltpu.async_copy(table_hbm.at[idx_v], rows_v, sem).wait()
    pltpu.sync_copy(rows_v, out_hbm.at[pl.ds(base, _ROWS_PER_WORKER)])

  return embed


def kernel(labels, table):
  return _make_embed(table.shape[0])(labels.astype(jnp.int32), table)
